# baseline (speedup 1.0000x reference)
import functools

import jax
import jax.numpy as jnp
from jax import lax
from jax.experimental import pallas as pl
from jax.experimental.pallas import tpu as pltpu
from jax.experimental.pallas import tpu_sc as plsc

NC, NS, L = 2, 16, 16
NW = NC * NS
B = 16384
D = 64
RPW = B // NW
CH = 128
NCHUNK = RPW // CH
CPAD = 1024

_EPS = 1e-12


def _rsqrt_vec(x):
    i = plsc.bitcast(x, jnp.int32)
    y = plsc.bitcast(jnp.int32(0x5F3759DF) - (i >> 1), jnp.float32)
    hx = x * jnp.float32(0.5)
    for _ in range(3):
        y = y * (jnp.float32(1.5) - (hx * y) * y)
    return y


def _prep_body(d_ref, r_ref, e_ref, c_ref):
    d = d_ref[...]
    r = r_ref[...]
    dn = jnp.maximum(jnp.sqrt(jnp.sum(d * d, axis=1, keepdims=True)), _EPS)
    rn = jnp.maximum(jnp.sqrt(jnp.sum(r * r, axis=1, keepdims=True)), _EPS)
    e = r / rn - d / dn
    e_ref[...] = e
    c_ref[...] = jnp.sum(e * e, axis=1)


@functools.partial(
    pl.kernel,
    out_type=jax.ShapeDtypeStruct((B,), jnp.float32),
    mesh=plsc.VectorSubcoreMesh(core_axis_name="c", subcore_axis_name="s"),
    scratch_types=[
        pltpu.VMEM((RPW,), jnp.int32),
        pltpu.VMEM((RPW,), jnp.int32),
        pltpu.VMEM((RPW, D), jnp.float32),
        pltpu.VMEM((RPW, D), jnp.float32),
        pltpu.VMEM((CPAD,), jnp.float32),
        pltpu.VMEM((RPW,), jnp.float32),
        pltpu.SemaphoreType.DMA,
    ],
    compiler_params=pltpu.CompilerParams(needs_layout_passes=False,
                                         use_tc_tiling_on_sc=False),
)
def _sc_main(gene_idx, drug_idx, gene_tab, e_tab, c_tab, out_hbm,
             gidx_v, didx_v, g_v, e_v, c_v, out_v, sem):
    wid = lax.axis_index("s") * NC + lax.axis_index("c")
    base = wid * RPW
    pltpu.sync_copy(gene_idx.at[pl.ds(base, RPW)], gidx_v)
    pltpu.sync_copy(drug_idx.at[pl.ds(base, RPW)], didx_v)
    pltpu.sync_copy(c_tab, c_v)
    copies = []
    for k in range(NCHUNK):
        copies.append(pltpu.async_copy(
            gene_tab.at[gidx_v.at[pl.ds(k * CH, CH)]],
            g_v.at[pl.ds(k * CH, CH)], sem))
        copies.append(pltpu.async_copy(
            e_tab.at[didx_v.at[pl.ds(k * CH, CH)]],
            e_v.at[pl.ds(k * CH, CH)], sem))
    for cp in copies:
        cp.wait()

    iota = lax.iota(jnp.int32, L)

    def grp(t, carry):
        rows = t * L + iota
        gg = jnp.zeros((L,), jnp.float32)
        ge = jnp.zeros((L,), jnp.float32)
        for j in range(D):
            cols = jnp.full((L,), j, jnp.int32)
            g = plsc.load_gather(g_v, [rows, cols])
            e = plsc.load_gather(e_v, [rows, cols])
            gg = gg + g * g
            ge = ge + g * e
        di = didx_v[pl.ds(t * L, L)]
        cc = plsc.load_gather(c_v, [di])
        s = jnp.maximum(gg * _rsqrt_vec(gg), jnp.float32(_EPS))
        inv = jnp.float32(1.0) / s
        tot = jnp.maximum(gg * inv * inv + (jnp.float32(2.0) * inv) * ge + cc,
                          jnp.float32(0.0))
        out_v[pl.ds(t * L, L)] = jnp.float32(0.0) - tot * _rsqrt_vec(tot)
        return carry

    lax.fori_loop(0, RPW // L, grp, 0)
    pltpu.sync_copy(out_v, out_hbm.at[pl.ds(base, RPW)])


def kernel(gene_idx, drug_idx, gene_table, drug_table, relation):
    gene_idx = gene_idx.astype(jnp.int32)
    drug_idx = drug_idx.astype(jnp.int32)
    nd = drug_table.shape[0]
    d_pad = jnp.pad(drug_table, ((0, CPAD - nd), (0, 0)), constant_values=1.0)
    e_tab, c_tab = pl.pallas_call(
        _prep_body,
        out_shape=[
            jax.ShapeDtypeStruct((CPAD, D), jnp.float32),
            jax.ShapeDtypeStruct((CPAD,), jnp.float32),
        ],
    )(d_pad, relation.reshape(1, D))
    return _sc_main(gene_idx, drug_idx, gene_table, e_tab, c_tab)


# speedup vs baseline: 1.0376x; 1.0376x over previous
import functools

import jax
import jax.numpy as jnp
from jax import lax
from jax.experimental import pallas as pl
from jax.experimental.pallas import tpu as pltpu
from jax.experimental.pallas import tpu_sc as plsc

NC, NS, L = 2, 16, 16
NW = NC * NS
B = 16384
D = 64
RPW = B // NW
CH = 128
NCHUNK = RPW // CH
CPAD = 1024

_EPS = 1e-12


def _rsqrt_vec(x):
    i = plsc.bitcast(x, jnp.int32)
    y = plsc.bitcast(jnp.int32(0x5F3759DF) - (i >> 1), jnp.float32)
    hx = x * jnp.float32(0.5)
    for _ in range(3):
        y = y * (jnp.float32(1.5) - (hx * y) * y)
    return y


def _prep_body(d_ref, r_ref, e_ref, c_ref):
    d = d_ref[...]
    r = r_ref[...]
    dn = jnp.maximum(jnp.sqrt(jnp.sum(d * d, axis=1, keepdims=True)), _EPS)
    rn = jnp.maximum(jnp.sqrt(jnp.sum(r * r, axis=1, keepdims=True)), _EPS)
    e = r / rn - d / dn
    e_ref[...] = e
    c_ref[...] = jnp.sum(e * e, axis=1)


@functools.partial(
    pl.kernel,
    out_type=jax.ShapeDtypeStruct((B,), jnp.float32),
    mesh=plsc.VectorSubcoreMesh(core_axis_name="c", subcore_axis_name="s"),
    scratch_types=[
        pltpu.VMEM((RPW,), jnp.int32),
        pltpu.VMEM((RPW,), jnp.int32),
        pltpu.VMEM((RPW, D), jnp.float32),
        pltpu.VMEM((RPW, D), jnp.float32),
        pltpu.VMEM((CPAD,), jnp.float32),
        pltpu.VMEM((RPW,), jnp.float32),
        pltpu.SemaphoreType.DMA,
    ],
    compiler_params=pltpu.CompilerParams(needs_layout_passes=False,
                                         use_tc_tiling_on_sc=False),
)
def _sc_main(gene_idx, drug_idx, gene_tab, e_tab, c_tab, out_hbm,
             gidx_v, didx_v, g_v, e_v, c_v, out_v, sem):
    wid = lax.axis_index("s") * NC + lax.axis_index("c")
    base = wid * RPW
    pltpu.sync_copy(gene_idx.at[pl.ds(base, RPW)], gidx_v)
    pltpu.sync_copy(drug_idx.at[pl.ds(base, RPW)], didx_v)
    pltpu.sync_copy(c_tab, c_v)
    copies = []
    for k in range(NCHUNK):
        copies.append(pltpu.async_copy(
            gene_tab.at[gidx_v.at[pl.ds(k * CH, CH)]],
            g_v.at[pl.ds(k * CH, CH)], sem))
        copies.append(pltpu.async_copy(
            e_tab.at[didx_v.at[pl.ds(k * CH, CH)]],
            e_v.at[pl.ds(k * CH, CH)], sem))
    for cp in copies:
        cp.wait()

    iota = lax.iota(jnp.int32, L)

    def grp(t, carry):
        rows = t * L + iota
        gg = jnp.zeros((L,), jnp.float32)
        ge = jnp.zeros((L,), jnp.float32)
        for j in range(D):
            cols = (iota + j) & (D - 1)
            g = plsc.load_gather(g_v, [rows, cols])
            e = plsc.load_gather(e_v, [rows, cols])
            gg = gg + g * g
            ge = ge + g * e
        di = didx_v[pl.ds(t * L, L)]
        cc = plsc.load_gather(c_v, [di])
        s = jnp.maximum(gg * _rsqrt_vec(gg), jnp.float32(_EPS))
        inv = jnp.float32(1.0) / s
        tot = jnp.maximum(gg * inv * inv + (jnp.float32(2.0) * inv) * ge + cc,
                          jnp.float32(0.0))
        out_v[pl.ds(t * L, L)] = jnp.float32(0.0) - tot * _rsqrt_vec(tot)
        return carry

    lax.fori_loop(0, RPW // L, grp, 0)
    pltpu.sync_copy(out_v, out_hbm.at[pl.ds(base, RPW)])


def kernel(gene_idx, drug_idx, gene_table, drug_table, relation):
    gene_idx = gene_idx.astype(jnp.int32)
    drug_idx = drug_idx.astype(jnp.int32)
    nd = drug_table.shape[0]
    d_pad = jnp.pad(drug_table, ((0, CPAD - nd), (0, 0)), constant_values=1.0)
    e_tab, c_tab = pl.pallas_call(
        _prep_body,
        out_shape=[
            jax.ShapeDtypeStruct((CPAD, D), jnp.float32),
            jax.ShapeDtypeStruct((CPAD,), jnp.float32),
        ],
    )(d_pad, relation.reshape(1, D))
    return _sc_main(gene_idx, drug_idx, gene_table, e_tab, c_tab)
